# baseline (speedup 1.0000x reference)
import functools

import jax
import jax.numpy as jnp
from jax import lax
from jax.experimental import pallas as pl
from jax.experimental.pallas import tpu as pltpu
from jax.experimental.pallas import tpu_sc as plsc

_NUM_WORKERS = 32
_LANES = 16

_V = 1000000
_VT = 7813
_VPAD = _VT * 128
_D = 64
_BATCH = 4096
_SEQ = 200
_N = _BATCH * _SEQ
_NBLK = _N // 128


def _wid():
  return lax.axis_index("s") * 2 + lax.axis_index("c")


def _iota16():
  return lax.iota(jnp.int32, 16)


def _make_transpose():
  mesh = plsc.VectorSubcoreMesh(core_axis_name="c", subcore_axis_name="s")
  cols_per_w = 246

  @functools.partial(
      pl.kernel,
      mesh=mesh,
      out_type=jax.ShapeDtypeStruct((_VPAD, 128), jnp.float32),
      scratch_types=[
          pltpu.VMEM((_D, 128), jnp.float32),
          pltpu.VMEM((_D, 128), jnp.float32),
          pltpu.VMEM((128, 128), jnp.float32),
          pltpu.VMEM((128, 128), jnp.float32),
          pltpu.SemaphoreType.DMA,
          pltpu.SemaphoreType.DMA,
          pltpu.SemaphoreType.DMA,
          pltpu.SemaphoreType.DMA,
      ],
      compiler_params=pltpu.CompilerParams(
          use_tc_tiling_on_sc=True, disable_bounds_checks=True,
          needs_layout_passes=False),
  )
  def transpose_kernel(et_hbm, out_hbm, in0, in1, tb0, tb1, g0, g1, s0, s1):
    base = _wid() * cols_per_w

    def col(i):
      return jnp.minimum(base + i, _VT - 1)

    def fire_read(i, buf, sem):
      pltpu.async_copy(et_hbm.at[:, pl.ds(col(i) * 128, 128)], buf, sem)

    def wait_read(buf, sem):
      pltpu.make_async_copy(et_hbm.at[:, pl.ds(0, 128)], buf, sem).wait()

    def transpose(src, dst):
      def body(br, carry):
        brv = jnp.full((16,), br, dtype=jnp.int32)
        for j in range(_D // _LANES):
          vals = plsc.load_gather(src, [_iota16() + j * _LANES, brv])
          dst[br, pl.ds(j * _LANES, _LANES)] = vals
        return carry

      lax.fori_loop(0, 128, body, 0)

    def fire_write(i, buf, sem):
      pltpu.async_copy(buf, out_hbm.at[pl.ds(col(i) * 128, 128)], sem)

    def wait_write(buf, sem):
      pltpu.make_async_copy(buf, out_hbm.at[pl.ds(0, 128)], sem).wait()

    fire_read(0, in0, g0)
    fire_read(1, in1, g1)

    def body(m, carry):
      i0 = 2 * m
      wait_read(in0, g0)

      @pl.when(m > 0)
      def _():
        wait_write(tb0, s0)

      transpose(in0, tb0)
      fire_write(i0, tb0, s0)

      @pl.when(m < cols_per_w // 2 - 1)
      def _():
        fire_read(i0 + 2, in0, g0)

      wait_read(in1, g1)

      @pl.when(m > 0)
      def _():
        wait_write(tb1, s1)

      transpose(in1, tb1)
      fire_write(i0 + 1, tb1, s1)

      @pl.when(m < cols_per_w // 2 - 1)
      def _():
        fire_read(i0 + 3, in1, g1)

      return carry

    lax.fori_loop(0, cols_per_w // 2, body, 0)
    wait_write(tb0, s0)
    wait_write(tb1, s1)

  return transpose_kernel


def _make_gather():
  mesh = plsc.VectorSubcoreMesh(core_axis_name="c", subcore_axis_name="s")
  blks_per_w = _NBLK // _NUM_WORKERS

  @functools.partial(
      pl.kernel,
      mesh=mesh,
      out_type=jax.ShapeDtypeStruct((_SEQ, 8, 32, 8, 128), jnp.float32),
      scratch_types=[
          pltpu.VMEM((blks_per_w, 128), jnp.int32),
          pltpu.VMEM((128, 128), jnp.float32),
          pltpu.VMEM((128, 128), jnp.float32),
          pltpu.VMEM((8, 8, 128), jnp.float32),
          pltpu.VMEM((8, 8, 128), jnp.float32),
          pltpu.SemaphoreType.DMA,
          pltpu.SemaphoreType.DMA,
          pltpu.SemaphoreType.DMA,
          pltpu.SemaphoreType.DMA,
          pltpu.SemaphoreType.DMA,
      ],
      compiler_params=pltpu.CompilerParams(
          use_tc_tiling_on_sc=False, needs_layout_passes=False),
  )
  def gather_kernel(ids_hbm, table_hbm, out_hbm, idx_v, r0, r1, t0, t1,
                    gi, g0, g1, s0, s1):
    base = _wid() * blks_per_w

    pltpu.sync_copy(ids_hbm.at[pl.ds(base, blks_per_w)], idx_v)

    def fire_gather(i, buf, sem):
      pltpu.async_copy(table_hbm.at[idx_v.at[i]], buf, sem)

    def wait_gather(buf, sem):
      pltpu.make_async_copy(table_hbm.at[idx_v.at[0]], buf, sem).wait()

    def transpose(src, dst):
      def body(f, carry):
        fv = jnp.full((16,), f, dtype=jnp.int32)
        ft = f // 8
        fr = f % 8
        for j in range(128 // _LANES):
          vals = plsc.load_gather(src, [_iota16() + j * _LANES, fv])
          dst[ft, fr, pl.ds(j * _LANES, _LANES)] = vals
        return carry

      lax.fori_loop(0, _D, body, 0)

    def fire_store(i, buf, sem):
      r = base + i
      s = r // 32
      k = r % 32
      pltpu.async_copy(buf, out_hbm.at[s, :, k], sem)

    def wait_store(buf, sem):
      pltpu.make_async_copy(buf, out_hbm.at[0, :, 0], sem).wait()

    fire_gather(0, r0, g0)
    fire_gather(1, r1, g1)

    def body(m, carry):
      i0 = 2 * m
      wait_gather(r0, g0)

      @pl.when(m > 0)
      def _():
        wait_store(t0, s0)

      transpose(r0, t0)
      fire_store(i0, t0, s0)

      @pl.when(m < blks_per_w // 2 - 1)
      def _():
        fire_gather(i0 + 2, r0, g0)

      wait_gather(r1, g1)

      @pl.when(m > 0)
      def _():
        wait_store(t1, s1)

      transpose(r1, t1)
      fire_store(i0 + 1, t1, s1)

      @pl.when(m < blks_per_w // 2 - 1)
      def _():
        fire_gather(i0 + 3, r1, g1)

      return carry

    lax.fori_loop(0, blks_per_w // 2, body, 0)
    wait_store(t0, s0)
    wait_store(t1, s1)

  return gather_kernel


def kernel(token_ids, E):
  ids = token_ids.T.reshape(_NBLK, 128).astype(jnp.int32)
  table = _make_transpose()(E.T)
  out5 = _make_gather()(ids, table)
  return out5.transpose(2, 4, 0, 1, 3).reshape(_BATCH, _SEQ, _D)


# speedup vs baseline: 1.0022x; 1.0022x over previous
import functools

import jax
import jax.numpy as jnp
from jax import lax
from jax.experimental import pallas as pl
from jax.experimental.pallas import tpu as pltpu
from jax.experimental.pallas import tpu_sc as plsc

_NUM_WORKERS = 32
_LANES = 16

_V = 1000000
_VT = 7813
_VPAD = _VT * 128
_D = 64
_BATCH = 4096
_SEQ = 200
_N = _BATCH * _SEQ
_NBLK = _N // 128


def _wid():
  return lax.axis_index("s") * 2 + lax.axis_index("c")


def _iota16():
  return lax.iota(jnp.int32, 16)


def _make_transpose():
  mesh = plsc.VectorSubcoreMesh(core_axis_name="c", subcore_axis_name="s")
  cols_per_w = 246

  @functools.partial(
      pl.kernel,
      mesh=mesh,
      out_type=jax.ShapeDtypeStruct((_VPAD, 128), jnp.float32),
      scratch_types=[
          pltpu.VMEM((_D, 128), jnp.float32),
          pltpu.VMEM((_D, 128), jnp.float32),
          pltpu.VMEM((128, 128), jnp.float32),
          pltpu.VMEM((128, 128), jnp.float32),
          pltpu.SemaphoreType.DMA,
          pltpu.SemaphoreType.DMA,
          pltpu.SemaphoreType.DMA,
          pltpu.SemaphoreType.DMA,
      ],
      compiler_params=pltpu.CompilerParams(
          use_tc_tiling_on_sc=True, disable_bounds_checks=True,
          needs_layout_passes=False),
  )
  def transpose_kernel(et_hbm, out_hbm, in0, in1, tb0, tb1, g0, g1, s0, s1):
    base = _wid() * cols_per_w

    def col(i):
      return jnp.minimum(base + i, _VT - 1)

    def fire_read(i, buf, sem):
      pltpu.async_copy(et_hbm.at[:, pl.ds(col(i) * 128, 128)], buf, sem)

    def wait_read(buf, sem):
      pltpu.make_async_copy(et_hbm.at[:, pl.ds(0, 128)], buf, sem).wait()

    def transpose(src, dst):
      rows = [_iota16() + j * _LANES for j in range(_D // _LANES)]

      def body(q, carry):
        for u in range(4):
          br = 4 * q + u
          brv = jnp.full((16,), br, dtype=jnp.int32)
          for j in range(_D // _LANES):
            vals = plsc.load_gather(src, [rows[j], brv])
            dst[br, pl.ds(j * _LANES, _LANES)] = vals
        return carry

      lax.fori_loop(0, 32, body, 0)

    def fire_write(i, buf, sem):
      pltpu.async_copy(buf, out_hbm.at[pl.ds(col(i) * 128, 128)], sem)

    def wait_write(buf, sem):
      pltpu.make_async_copy(buf, out_hbm.at[pl.ds(0, 128)], sem).wait()

    fire_read(0, in0, g0)
    fire_read(1, in1, g1)

    def body(m, carry):
      i0 = 2 * m
      wait_read(in0, g0)

      @pl.when(m > 0)
      def _():
        wait_write(tb0, s0)

      transpose(in0, tb0)
      fire_write(i0, tb0, s0)

      @pl.when(m < cols_per_w // 2 - 1)
      def _():
        fire_read(i0 + 2, in0, g0)

      wait_read(in1, g1)

      @pl.when(m > 0)
      def _():
        wait_write(tb1, s1)

      transpose(in1, tb1)
      fire_write(i0 + 1, tb1, s1)

      @pl.when(m < cols_per_w // 2 - 1)
      def _():
        fire_read(i0 + 3, in1, g1)

      return carry

    lax.fori_loop(0, cols_per_w // 2, body, 0)
    wait_write(tb0, s0)
    wait_write(tb1, s1)

  return transpose_kernel


def _make_gather():
  mesh = plsc.VectorSubcoreMesh(core_axis_name="c", subcore_axis_name="s")
  blks_per_w = _NBLK // _NUM_WORKERS

  @functools.partial(
      pl.kernel,
      mesh=mesh,
      out_type=jax.ShapeDtypeStruct((_SEQ, 8, 32, 8, 128), jnp.float32),
      scratch_types=[
          pltpu.VMEM((blks_per_w, 128), jnp.int32),
          pltpu.VMEM((128, _D), jnp.float32),
          pltpu.VMEM((128, _D), jnp.float32),
          pltpu.VMEM((8, 8, 128), jnp.float32),
          pltpu.VMEM((8, 8, 128), jnp.float32),
          pltpu.SemaphoreType.DMA,
          pltpu.SemaphoreType.DMA,
          pltpu.SemaphoreType.DMA,
          pltpu.SemaphoreType.DMA,
          pltpu.SemaphoreType.DMA,
      ],
      compiler_params=pltpu.CompilerParams(
          use_tc_tiling_on_sc=False, needs_layout_passes=False),
  )
  def gather_kernel(ids_hbm, table_hbm, out_hbm, idx_v, r0, r1, t0, t1,
                    gi, g0, g1, s0, s1):
    base = _wid() * blks_per_w

    pltpu.sync_copy(ids_hbm.at[pl.ds(base, blks_per_w)], idx_v)

    def fire_gather(i, buf, sem):
      pltpu.async_copy(table_hbm.at[idx_v.at[i]], buf, sem)

    def wait_gather(buf, sem):
      pltpu.make_async_copy(table_hbm.at[idx_v.at[0]], buf, sem).wait()

    def transpose(src, dst):
      rows = [_iota16() + j * _LANES for j in range(128 // _LANES)]

      def body(q, carry):
        f0 = 4 * q
        ft = f0 // 8
        for u in range(4):
          f = f0 + u
          fv = jnp.full((16,), f, dtype=jnp.int32)
          fr = f - 8 * ft
          for j in range(128 // _LANES):
            vals = plsc.load_gather(src, [rows[j], fv])
            dst[ft, fr, pl.ds(j * _LANES, _LANES)] = vals
        return carry

      lax.fori_loop(0, _D // 4, body, 0)

    def fire_store(i, buf, sem):
      r = base + i
      s = r // 32
      k = r % 32
      pltpu.async_copy(buf, out_hbm.at[s, :, k], sem)

    def wait_store(buf, sem):
      pltpu.make_async_copy(buf, out_hbm.at[0, :, 0], sem).wait()

    fire_gather(0, r0, g0)
    fire_gather(1, r1, g1)

    def body(m, carry):
      i0 = 2 * m
      wait_gather(r0, g0)

      @pl.when(m > 0)
      def _():
        wait_store(t0, s0)

      transpose(r0, t0)
      fire_store(i0, t0, s0)

      @pl.when(m < blks_per_w // 2 - 1)
      def _():
        fire_gather(i0 + 2, r0, g0)

      wait_gather(r1, g1)

      @pl.when(m > 0)
      def _():
        wait_store(t1, s1)

      transpose(r1, t1)
      fire_store(i0 + 1, t1, s1)

      @pl.when(m < blks_per_w // 2 - 1)
      def _():
        fire_gather(i0 + 3, r1, g1)

      return carry

    lax.fori_loop(0, blks_per_w // 2, body, 0)
    wait_store(t0, s0)
    wait_store(t1, s1)

  return gather_kernel


def kernel(token_ids, E):
  ids = token_ids.T.reshape(_NBLK, 128).astype(jnp.int32) * 2
  table = _make_transpose()(E.T).reshape(2 * _VPAD, _D)
  out5 = _make_gather()(ids, table)
  return out5.transpose(2, 4, 0, 1, 3).reshape(_BATCH, _SEQ, _D)


# speedup vs baseline: 1.8474x; 1.8434x over previous
import functools

import jax
import jax.numpy as jnp
from jax import lax
from jax.experimental import pallas as pl
from jax.experimental.pallas import tpu as pltpu
from jax.experimental.pallas import tpu_sc as plsc

_NUM_WORKERS = 32
_LANES = 16

_V = 1000000
_VT = 7813
_VPAD = _VT * 128
_D = 64
_BATCH = 4096
_SEQ = 200
_N = _BATCH * _SEQ
_NBLK = _N // 128


def _wid():
  return lax.axis_index("s") * 2 + lax.axis_index("c")


def _iota16():
  return lax.iota(jnp.int32, 16)


def _make_transpose():
  mesh = plsc.VectorSubcoreMesh(core_axis_name="c", subcore_axis_name="s")
  cols_per_w = 246

  @functools.partial(
      pl.kernel,
      mesh=mesh,
      out_type=jax.ShapeDtypeStruct((_VPAD, 128), jnp.float32),
      scratch_types=[
          pltpu.VMEM((_D, 128), jnp.float32),
          pltpu.VMEM((_D, 128), jnp.float32),
          pltpu.VMEM((128, 128), jnp.float32),
          pltpu.VMEM((128, 128), jnp.float32),
          pltpu.SemaphoreType.DMA,
          pltpu.SemaphoreType.DMA,
          pltpu.SemaphoreType.DMA,
          pltpu.SemaphoreType.DMA,
      ],
      compiler_params=pltpu.CompilerParams(
          use_tc_tiling_on_sc=True, disable_bounds_checks=True,
          needs_layout_passes=False),
  )
  def transpose_kernel(et_hbm, out_hbm, in0, in1, tb0, tb1, g0, g1, s0, s1):
    base = _wid() * cols_per_w

    def col(i):
      return jnp.minimum(base + i, _VT - 1)

    def fire_read(i, buf, sem):
      pltpu.async_copy(et_hbm.at[:, pl.ds(col(i) * 128, 128)], buf, sem)

    def wait_read(buf, sem):
      pltpu.make_async_copy(et_hbm.at[:, pl.ds(0, 128)], buf, sem).wait()

    def transpose(src, dst):
      rows = [_iota16() + j * _LANES for j in range(_D // _LANES)]

      @plsc.parallel_loop(0, 128, 1, unroll=8)
      def _(br):
        brv = jnp.full((16,), br, dtype=jnp.int32)
        for j in range(_D // _LANES):
          dst[br, pl.ds(j * _LANES, _LANES)] = plsc.load_gather(
              src, [rows[j], brv])

    def fire_write(i, buf, sem):
      pltpu.async_copy(buf, out_hbm.at[pl.ds(col(i) * 128, 128)], sem)

    def wait_write(buf, sem):
      pltpu.make_async_copy(buf, out_hbm.at[pl.ds(0, 128)], sem).wait()

    fire_read(0, in0, g0)
    fire_read(1, in1, g1)

    def body(m, carry):
      i0 = 2 * m
      wait_read(in0, g0)

      @pl.when(m > 0)
      def _():
        wait_write(tb0, s0)

      transpose(in0, tb0)
      fire_write(i0, tb0, s0)

      @pl.when(m < cols_per_w // 2 - 1)
      def _():
        fire_read(i0 + 2, in0, g0)

      wait_read(in1, g1)

      @pl.when(m > 0)
      def _():
        wait_write(tb1, s1)

      transpose(in1, tb1)
      fire_write(i0 + 1, tb1, s1)

      @pl.when(m < cols_per_w // 2 - 1)
      def _():
        fire_read(i0 + 3, in1, g1)

      return carry

    lax.fori_loop(0, cols_per_w // 2, body, 0)
    wait_write(tb0, s0)
    wait_write(tb1, s1)

  return transpose_kernel


def _make_gather():
  mesh = plsc.VectorSubcoreMesh(core_axis_name="c", subcore_axis_name="s")
  blks_per_w = _NBLK // _NUM_WORKERS

  @functools.partial(
      pl.kernel,
      mesh=mesh,
      out_type=jax.ShapeDtypeStruct((_SEQ, 8, 32, 8, 128), jnp.float32),
      scratch_types=[
          pltpu.VMEM((blks_per_w, 128), jnp.int32),
          pltpu.VMEM((128, _D), jnp.float32),
          pltpu.VMEM((128, _D), jnp.float32),
          pltpu.VMEM((8, 8, 128), jnp.float32),
          pltpu.VMEM((8, 8, 128), jnp.float32),
          pltpu.SemaphoreType.DMA,
          pltpu.SemaphoreType.DMA,
          pltpu.SemaphoreType.DMA,
          pltpu.SemaphoreType.DMA,
          pltpu.SemaphoreType.DMA,
      ],
      compiler_params=pltpu.CompilerParams(
          use_tc_tiling_on_sc=False, needs_layout_passes=False),
  )
  def gather_kernel(ids_hbm, table_hbm, out_hbm, idx_v, r0, r1, t0, t1,
                    gi, g0, g1, s0, s1):
    base = _wid() * blks_per_w

    pltpu.sync_copy(ids_hbm.at[pl.ds(base, blks_per_w)], idx_v)

    def fire_gather(i, buf, sem):
      pltpu.async_copy(table_hbm.at[idx_v.at[i]], buf, sem)

    def wait_gather(buf, sem):
      pltpu.make_async_copy(table_hbm.at[idx_v.at[0]], buf, sem).wait()

    def transpose(src, dst):
      rows = [_iota16() + j * _LANES for j in range(128 // _LANES)]

      @plsc.parallel_loop(0, _D, 1, unroll=8)
      def _(f):
        fv = jnp.full((16,), f, dtype=jnp.int32)
        ft = f // 8
        fr = f % 8
        for j in range(128 // _LANES):
          dst[ft, fr, pl.ds(j * _LANES, _LANES)] = plsc.load_gather(
              src, [rows[j], fv])

    def fire_store(i, buf, sem):
      r = base + i
      s = r // 32
      k = r % 32
      pltpu.async_copy(buf, out_hbm.at[s, :, k], sem)

    def wait_store(buf, sem):
      pltpu.make_async_copy(buf, out_hbm.at[0, :, 0], sem).wait()

    fire_gather(0, r0, g0)
    fire_gather(1, r1, g1)

    def body(m, carry):
      i0 = 2 * m
      wait_gather(r0, g0)

      @pl.when(m > 0)
      def _():
        wait_store(t0, s0)

      transpose(r0, t0)
      fire_store(i0, t0, s0)

      @pl.when(m < blks_per_w // 2 - 1)
      def _():
        fire_gather(i0 + 2, r0, g0)

      wait_gather(r1, g1)

      @pl.when(m > 0)
      def _():
        wait_store(t1, s1)

      transpose(r1, t1)
      fire_store(i0 + 1, t1, s1)

      @pl.when(m < blks_per_w // 2 - 1)
      def _():
        fire_gather(i0 + 3, r1, g1)

      return carry

    lax.fori_loop(0, blks_per_w // 2, body, 0)
    wait_store(t0, s0)
    wait_store(t1, s1)

  return gather_kernel


def kernel(token_ids, E):
  ids = token_ids.T.reshape(_NBLK, 128).astype(jnp.int32) * 2
  table = _make_transpose()(E.T).reshape(2 * _VPAD, _D)
  out5 = _make_gather()(ids, table)
  return out5.transpose(2, 4, 0, 1, 3).reshape(_BATCH, _SEQ, _D)


# speedup vs baseline: 1.8551x; 1.0041x over previous
import functools

import jax
import jax.numpy as jnp
from jax import lax
from jax.experimental import pallas as pl
from jax.experimental.pallas import tpu as pltpu
from jax.experimental.pallas import tpu_sc as plsc

_NUM_WORKERS = 32
_LANES = 16

_V = 1000000
_VT = 7813
_VPAD = _VT * 128
_D = 64
_BATCH = 4096
_SEQ = 200
_N = _BATCH * _SEQ
_NBLK = _N // 128


def _wid():
  return lax.axis_index("s") * 2 + lax.axis_index("c")


def _iota16():
  return lax.iota(jnp.int32, 16)


def _make_transpose():
  mesh = plsc.VectorSubcoreMesh(core_axis_name="c", subcore_axis_name="s")
  cols_per_w = 246

  @functools.partial(
      pl.kernel,
      mesh=mesh,
      out_type=jax.ShapeDtypeStruct((_VPAD, 128), jnp.float32),
      scratch_types=[
          pltpu.VMEM((_D, 128), jnp.float32),
          pltpu.VMEM((_D, 128), jnp.float32),
          pltpu.VMEM((128, 128), jnp.float32),
          pltpu.VMEM((128, 128), jnp.float32),
          pltpu.SemaphoreType.DMA,
          pltpu.SemaphoreType.DMA,
          pltpu.SemaphoreType.DMA,
          pltpu.SemaphoreType.DMA,
      ],
      compiler_params=pltpu.CompilerParams(
          use_tc_tiling_on_sc=True, disable_bounds_checks=True,
          needs_layout_passes=False),
  )
  def transpose_kernel(et_hbm, out_hbm, in0, in1, tb0, tb1, g0, g1, s0, s1):
    base = _wid() * cols_per_w

    def col(i):
      return jnp.minimum(base + i, _VT - 1)

    def fire_read(i, buf, sem):
      pltpu.async_copy(et_hbm.at[:, pl.ds(col(i) * 128, 128)], buf, sem)

    def wait_read(buf, sem):
      pltpu.make_async_copy(et_hbm.at[:, pl.ds(0, 128)], buf, sem).wait()

    def transpose(src, dst):
      rows = [_iota16() + j * _LANES for j in range(_D // _LANES)]

      @plsc.parallel_loop(0, 128, 1, unroll=16)
      def _(br):
        brv = jnp.full((16,), br, dtype=jnp.int32)
        for j in range(_D // _LANES):
          dst[br, pl.ds(j * _LANES, _LANES)] = plsc.load_gather(
              src, [rows[j], brv])

    def fire_write(i, buf, sem):
      pltpu.async_copy(buf, out_hbm.at[pl.ds(col(i) * 128, 128)], sem)

    def wait_write(buf, sem):
      pltpu.make_async_copy(buf, out_hbm.at[pl.ds(0, 128)], sem).wait()

    fire_read(0, in0, g0)
    fire_read(1, in1, g1)

    def body(m, carry):
      i0 = 2 * m
      wait_read(in0, g0)

      @pl.when(m > 0)
      def _():
        wait_write(tb0, s0)

      transpose(in0, tb0)
      fire_write(i0, tb0, s0)

      @pl.when(m < cols_per_w // 2 - 1)
      def _():
        fire_read(i0 + 2, in0, g0)

      wait_read(in1, g1)

      @pl.when(m > 0)
      def _():
        wait_write(tb1, s1)

      transpose(in1, tb1)
      fire_write(i0 + 1, tb1, s1)

      @pl.when(m < cols_per_w // 2 - 1)
      def _():
        fire_read(i0 + 3, in1, g1)

      return carry

    lax.fori_loop(0, cols_per_w // 2, body, 0)
    wait_write(tb0, s0)
    wait_write(tb1, s1)

  return transpose_kernel


def _make_gather():
  mesh = plsc.VectorSubcoreMesh(core_axis_name="c", subcore_axis_name="s")
  blks_per_w = _NBLK // _NUM_WORKERS

  @functools.partial(
      pl.kernel,
      mesh=mesh,
      out_type=jax.ShapeDtypeStruct((_SEQ, 8, 32, 8, 128), jnp.float32),
      scratch_types=[
          pltpu.VMEM((blks_per_w, 128), jnp.int32),
          pltpu.VMEM((128, _D), jnp.float32),
          pltpu.VMEM((128, _D), jnp.float32),
          pltpu.VMEM((8, 8, 128), jnp.float32),
          pltpu.VMEM((8, 8, 128), jnp.float32),
          pltpu.SemaphoreType.DMA,
          pltpu.SemaphoreType.DMA,
          pltpu.SemaphoreType.DMA,
          pltpu.SemaphoreType.DMA,
          pltpu.SemaphoreType.DMA,
      ],
      compiler_params=pltpu.CompilerParams(
          use_tc_tiling_on_sc=False, needs_layout_passes=False),
  )
  def gather_kernel(ids_hbm, table_hbm, out_hbm, idx_v, r0, r1, t0, t1,
                    gi, g0, g1, s0, s1):
    base = _wid() * blks_per_w

    pltpu.sync_copy(ids_hbm.at[pl.ds(base, blks_per_w)], idx_v)

    def fire_gather(i, buf, sem):
      pltpu.async_copy(table_hbm.at[idx_v.at[i]], buf, sem)

    def wait_gather(buf, sem):
      pltpu.make_async_copy(table_hbm.at[idx_v.at[0]], buf, sem).wait()

    def transpose(src, dst):
      rows = [_iota16() + j * _LANES for j in range(128 // _LANES)]

      @plsc.parallel_loop(0, _D, 1, unroll=16)
      def _(f):
        fv = jnp.full((16,), f, dtype=jnp.int32)
        ft = f // 8
        fr = f % 8
        for j in range(128 // _LANES):
          dst[ft, fr, pl.ds(j * _LANES, _LANES)] = plsc.load_gather(
              src, [rows[j], fv])

    def fire_store(i, buf, sem):
      r = base + i
      s = r // 32
      k = r % 32
      pltpu.async_copy(buf, out_hbm.at[s, :, k], sem)

    def wait_store(buf, sem):
      pltpu.make_async_copy(buf, out_hbm.at[0, :, 0], sem).wait()

    fire_gather(0, r0, g0)
    fire_gather(1, r1, g1)

    def body(m, carry):
      i0 = 2 * m
      wait_gather(r0, g0)

      @pl.when(m > 0)
      def _():
        wait_store(t0, s0)

      transpose(r0, t0)
      fire_store(i0, t0, s0)

      @pl.when(m < blks_per_w // 2 - 1)
      def _():
        fire_gather(i0 + 2, r0, g0)

      wait_gather(r1, g1)

      @pl.when(m > 0)
      def _():
        wait_store(t1, s1)

      transpose(r1, t1)
      fire_store(i0 + 1, t1, s1)

      @pl.when(m < blks_per_w // 2 - 1)
      def _():
        fire_gather(i0 + 3, r1, g1)

      return carry

    lax.fori_loop(0, blks_per_w // 2, body, 0)
    wait_store(t0, s0)
    wait_store(t1, s1)

  return gather_kernel


def kernel(token_ids, E):
  ids = token_ids.T.reshape(_NBLK, 128).astype(jnp.int32) * 2
  table = _make_transpose()(E.T).reshape(2 * _VPAD, _D)
  out5 = _make_gather()(ids, table)
  return out5.transpose(2, 4, 0, 1, 3).reshape(_BATCH, _SEQ, _D)


# speedup vs baseline: 2.5802x; 1.3909x over previous
import functools

import jax
import jax.numpy as jnp
from jax import lax
from jax.experimental import pallas as pl
from jax.experimental.pallas import tpu as pltpu
from jax.experimental.pallas import tpu_sc as plsc

_NUM_WORKERS = 32
_LANES = 16

_V = 1000000
_VT = 7813
_VPAD = _VT * 128
_D = 64
_BATCH = 4096
_SEQ = 200
_N = _BATCH * _SEQ
_NBLK = _N // 128


def _wid():
  return lax.axis_index("s") * 2 + lax.axis_index("c")


def _iota16():
  return lax.iota(jnp.int32, 16)


def _make_transpose():
  mesh = plsc.VectorSubcoreMesh(core_axis_name="c", subcore_axis_name="s")
  cols_per_w = 246

  @functools.partial(
      pl.kernel,
      mesh=mesh,
      out_type=jax.ShapeDtypeStruct((_VPAD, 128), jnp.float32),
      scratch_types=[
          pltpu.VMEM((_D, 128), jnp.float32),
          pltpu.VMEM((_D, 128), jnp.float32),
          pltpu.VMEM((128, 129), jnp.float32),
          pltpu.VMEM((128, 129), jnp.float32),
          pltpu.SemaphoreType.DMA,
          pltpu.SemaphoreType.DMA,
          pltpu.SemaphoreType.DMA,
          pltpu.SemaphoreType.DMA,
      ],
      compiler_params=pltpu.CompilerParams(
          use_tc_tiling_on_sc=True, disable_bounds_checks=True,
          needs_layout_passes=False),
  )
  def transpose_kernel(et_hbm, out_hbm, in0, in1, tb0, tb1, g0, g1, s0, s1):
    base = _wid() * cols_per_w

    def col(i):
      return jnp.minimum(base + i, _VT - 1)

    def fire_read(i, buf, sem):
      pltpu.async_copy(et_hbm.at[:, pl.ds(col(i) * 128, 128)], buf, sem)

    def wait_read(buf, sem):
      pltpu.make_async_copy(et_hbm.at[:, pl.ds(0, 128)], buf, sem).wait()

    def transpose(src, dst):
      rows = [_iota16() + j * _LANES for j in range(128 // _LANES)]

      @plsc.parallel_loop(0, _D, 1, unroll=8)
      def _(c):
        cv = jnp.full((16,), c, dtype=jnp.int32)
        for j in range(128 // _LANES):
          plsc.store_scatter(dst, [rows[j], cv],
                             src[c, pl.ds(j * _LANES, _LANES)])

    def fire_write(i, buf, sem):
      pltpu.async_copy(buf.at[:, pl.ds(0, 128)],
                       out_hbm.at[pl.ds(col(i) * 128, 128)], sem)

    def wait_write(buf, sem):
      pltpu.make_async_copy(buf.at[:, pl.ds(0, 128)],
                            out_hbm.at[pl.ds(0, 128)], sem).wait()

    fire_read(0, in0, g0)
    fire_read(1, in1, g1)

    def body(m, carry):
      i0 = 2 * m
      wait_read(in0, g0)

      @pl.when(m > 0)
      def _():
        wait_write(tb0, s0)

      transpose(in0, tb0)
      fire_write(i0, tb0, s0)

      @pl.when(m < cols_per_w // 2 - 1)
      def _():
        fire_read(i0 + 2, in0, g0)

      wait_read(in1, g1)

      @pl.when(m > 0)
      def _():
        wait_write(tb1, s1)

      transpose(in1, tb1)
      fire_write(i0 + 1, tb1, s1)

      @pl.when(m < cols_per_w // 2 - 1)
      def _():
        fire_read(i0 + 3, in1, g1)

      return carry

    lax.fori_loop(0, cols_per_w // 2, body, 0)
    wait_write(tb0, s0)
    wait_write(tb1, s1)

  return transpose_kernel


def _make_gather():
  mesh = plsc.VectorSubcoreMesh(core_axis_name="c", subcore_axis_name="s")
  blks_per_w = _NBLK // _NUM_WORKERS

  @functools.partial(
      pl.kernel,
      mesh=mesh,
      out_type=jax.ShapeDtypeStruct((_SEQ, 8, 32, 8, 128), jnp.float32),
      scratch_types=[
          pltpu.VMEM((blks_per_w, 128), jnp.int32),
          pltpu.VMEM((128, _D), jnp.float32),
          pltpu.VMEM((128, _D), jnp.float32),
          pltpu.VMEM((8, 8, 129), jnp.float32),
          pltpu.VMEM((8, 8, 129), jnp.float32),
          pltpu.SemaphoreType.DMA,
          pltpu.SemaphoreType.DMA,
          pltpu.SemaphoreType.DMA,
          pltpu.SemaphoreType.DMA,
          pltpu.SemaphoreType.DMA,
      ],
      compiler_params=pltpu.CompilerParams(
          use_tc_tiling_on_sc=False, needs_layout_passes=False),
  )
  def gather_kernel(ids_hbm, table_hbm, out_hbm, idx_v, r0, r1, t0, t1,
                    gi, g0, g1, s0, s1):
    base = _wid() * blks_per_w

    pltpu.sync_copy(ids_hbm.at[pl.ds(base, blks_per_w)], idx_v)

    def fire_gather(i, buf, sem):
      pltpu.async_copy(table_hbm.at[idx_v.at[i]], buf, sem)

    def wait_gather(buf, sem):
      pltpu.make_async_copy(table_hbm.at[idx_v.at[0]], buf, sem).wait()

    def transpose(src, dst):
      fts = [(_iota16() + j * _LANES) // 8 for j in range(_D // _LANES)]
      frs = [(_iota16() + j * _LANES) % 8 for j in range(_D // _LANES)]

      @plsc.parallel_loop(0, 128, 1, unroll=8)
      def _(br):
        brv = jnp.full((16,), br, dtype=jnp.int32)
        for j in range(_D // _LANES):
          plsc.store_scatter(dst, [fts[j], frs[j], brv],
                             src[br, pl.ds(j * _LANES, _LANES)])

    def fire_store(i, buf, sem):
      r = base + i
      s = r // 32
      k = r % 32
      pltpu.async_copy(buf.at[:, :, pl.ds(0, 128)], out_hbm.at[s, :, k], sem)

    def wait_store(buf, sem):
      pltpu.make_async_copy(buf.at[:, :, pl.ds(0, 128)],
                            out_hbm.at[0, :, 0], sem).wait()

    fire_gather(0, r0, g0)
    fire_gather(1, r1, g1)

    def body(m, carry):
      i0 = 2 * m
      wait_gather(r0, g0)

      @pl.when(m > 0)
      def _():
        wait_store(t0, s0)

      transpose(r0, t0)
      fire_store(i0, t0, s0)

      @pl.when(m < blks_per_w // 2 - 1)
      def _():
        fire_gather(i0 + 2, r0, g0)

      wait_gather(r1, g1)

      @pl.when(m > 0)
      def _():
        wait_store(t1, s1)

      transpose(r1, t1)
      fire_store(i0 + 1, t1, s1)

      @pl.when(m < blks_per_w // 2 - 1)
      def _():
        fire_gather(i0 + 3, r1, g1)

      return carry

    lax.fori_loop(0, blks_per_w // 2, body, 0)
    wait_store(t0, s0)
    wait_store(t1, s1)

  return gather_kernel


def kernel(token_ids, E):
  ids = token_ids.T.reshape(_NBLK, 128).astype(jnp.int32) * 2
  table = _make_transpose()(E.T).reshape(2 * _VPAD, _D)
  out5 = _make_gather()(ids, table)
  return out5.transpose(2, 4, 0, 1, 3).reshape(_BATCH, _SEQ, _D)


# speedup vs baseline: 2.6646x; 1.0327x over previous
import functools

import jax
import jax.numpy as jnp
from jax import lax
from jax.experimental import pallas as pl
from jax.experimental.pallas import tpu as pltpu
from jax.experimental.pallas import tpu_sc as plsc

_NUM_WORKERS = 32
_LANES = 16

_V = 1000000
_VT = 7813
_VPAD = _VT * 128
_D = 64
_BATCH = 4096
_SEQ = 200
_N = _BATCH * _SEQ
_NBLK = _N // 128


def _wid():
  return lax.axis_index("s") * 2 + lax.axis_index("c")


def _iota16():
  return lax.iota(jnp.int32, 16)


def _make_transpose():
  mesh = plsc.VectorSubcoreMesh(core_axis_name="c", subcore_axis_name="s")
  cols_per_w = 246

  @functools.partial(
      pl.kernel,
      mesh=mesh,
      out_type=jax.ShapeDtypeStruct((_VPAD, 128), jnp.float32),
      scratch_types=[
          pltpu.VMEM((_D, 128), jnp.float32),
          pltpu.VMEM((_D, 128), jnp.float32),
          pltpu.VMEM((128, 129), jnp.float32),
          pltpu.VMEM((128, 129), jnp.float32),
          pltpu.SemaphoreType.DMA,
          pltpu.SemaphoreType.DMA,
          pltpu.SemaphoreType.DMA,
          pltpu.SemaphoreType.DMA,
      ],
      compiler_params=pltpu.CompilerParams(
          use_tc_tiling_on_sc=True, disable_bounds_checks=True,
          needs_layout_passes=False),
  )
  def transpose_kernel(et_hbm, out_hbm, in0, in1, tb0, tb1, g0, g1, s0, s1):
    base = _wid() * cols_per_w

    def col(i):
      return jnp.minimum(base + i, _VT - 1)

    def fire_read(i, buf, sem):
      pltpu.async_copy(et_hbm.at[:, pl.ds(col(i) * 128, 128)], buf, sem)

    def wait_read(buf, sem):
      pltpu.make_async_copy(et_hbm.at[:, pl.ds(0, 128)], buf, sem).wait()

    def transpose(src, dst):
      rows = [_iota16() + j * _LANES for j in range(128 // _LANES)]

      @plsc.parallel_loop(0, _D, 1, unroll=8)
      def _(c):
        cv = jnp.full((16,), c, dtype=jnp.int32)
        for j in range(128 // _LANES):
          plsc.store_scatter(dst, [rows[j], cv],
                             src[c, pl.ds(j * _LANES, _LANES)])

    def fire_write(i, buf, sem):
      pltpu.async_copy(buf.at[:, pl.ds(0, 128)],
                       out_hbm.at[pl.ds(col(i) * 128, 128)], sem)

    def wait_write(buf, sem):
      pltpu.make_async_copy(buf.at[:, pl.ds(0, 128)],
                            out_hbm.at[pl.ds(0, 128)], sem).wait()

    fire_read(0, in0, g0)
    fire_read(1, in1, g1)

    def body(m, carry):
      i0 = 2 * m
      wait_read(in0, g0)

      @pl.when(m > 0)
      def _():
        wait_write(tb0, s0)

      transpose(in0, tb0)
      fire_write(i0, tb0, s0)

      @pl.when(m < cols_per_w // 2 - 1)
      def _():
        fire_read(i0 + 2, in0, g0)

      wait_read(in1, g1)

      @pl.when(m > 0)
      def _():
        wait_write(tb1, s1)

      transpose(in1, tb1)
      fire_write(i0 + 1, tb1, s1)

      @pl.when(m < cols_per_w // 2 - 1)
      def _():
        fire_read(i0 + 3, in1, g1)

      return carry

    lax.fori_loop(0, cols_per_w // 2, body, 0)
    wait_write(tb0, s0)
    wait_write(tb1, s1)

  return transpose_kernel


def _make_gather():
  mesh = plsc.VectorSubcoreMesh(core_axis_name="c", subcore_axis_name="s")
  blks_per_w = _NBLK // _NUM_WORKERS
  ring = 4

  @functools.partial(
      pl.kernel,
      mesh=mesh,
      out_type=jax.ShapeDtypeStruct((_SEQ, 8, 32, 8, 128), jnp.float32),
      scratch_types=(
          [pltpu.VMEM((blks_per_w, 128), jnp.int32)]
          + [pltpu.VMEM((128, _D), jnp.float32)] * ring
          + [pltpu.VMEM((8, 8, 129), jnp.float32)] * ring
          + [pltpu.SemaphoreType.DMA] * (2 * ring)
      ),
      compiler_params=pltpu.CompilerParams(
          use_tc_tiling_on_sc=False, needs_layout_passes=False),
  )
  def gather_kernel(ids_hbm, table_hbm, out_hbm, idx_v, *bufs):
    rb = bufs[0:ring]
    tb = bufs[ring:2 * ring]
    gsem = bufs[2 * ring:3 * ring]
    ssem = bufs[3 * ring:4 * ring]
    base = _wid() * blks_per_w

    pltpu.sync_copy(ids_hbm.at[pl.ds(base, blks_per_w)], idx_v)

    def fire_gather(i, u):
      pltpu.async_copy(table_hbm.at[idx_v.at[i]], rb[u], gsem[u])

    def wait_gather(u):
      pltpu.make_async_copy(table_hbm.at[idx_v.at[0]], rb[u],
                            gsem[u]).wait()

    def transpose(src, dst):
      fts = [(_iota16() + j * _LANES) // 8 for j in range(_D // _LANES)]
      frs = [(_iota16() + j * _LANES) % 8 for j in range(_D // _LANES)]

      @plsc.parallel_loop(0, 128, 1, unroll=8)
      def _(br):
        brv = jnp.full((16,), br, dtype=jnp.int32)
        for j in range(_D // _LANES):
          plsc.store_scatter(dst, [fts[j], frs[j], brv],
                             src[br, pl.ds(j * _LANES, _LANES)])

    def fire_store(i, u):
      r = base + i
      s = r // 32
      k = r % 32
      pltpu.async_copy(tb[u].at[:, :, pl.ds(0, 128)], out_hbm.at[s, :, k],
                       ssem[u])

    def wait_store(u):
      pltpu.make_async_copy(tb[u].at[:, :, pl.ds(0, 128)],
                            out_hbm.at[0, :, 0], ssem[u]).wait()

    for u in range(ring):
      fire_gather(u, u)

    def body(m, carry):
      i0 = ring * m
      for u in range(ring):
        wait_gather(u)

        @pl.when(m > 0)
        def _():
          wait_store(u)

        transpose(rb[u], tb[u])
        fire_store(i0 + u, u)

        @pl.when(m < blks_per_w // ring - 1)
        def _():
          fire_gather(i0 + ring + u, u)

      return carry

    lax.fori_loop(0, blks_per_w // ring, body, 0)
    for u in range(ring):
      wait_store(u)

  return gather_kernel


def kernel(token_ids, E):
  ids = token_ids.T.reshape(_NBLK, 128).astype(jnp.int32) * 2
  table = _make_transpose()(E.T).reshape(2 * _VPAD, _D)
  out5 = _make_gather()(ids, table)
  return out5.transpose(2, 4, 0, 1, 3).reshape(_BATCH, _SEQ, _D)


# speedup vs baseline: 2.6681x; 1.0013x over previous
import functools

import jax
import jax.numpy as jnp
from jax import lax
from jax.experimental import pallas as pl
from jax.experimental.pallas import tpu as pltpu
from jax.experimental.pallas import tpu_sc as plsc

_NUM_WORKERS = 32
_LANES = 16

_V = 1000000
_VT = 7813
_VPAD = _VT * 128
_D = 64
_BATCH = 4096
_SEQ = 200
_N = _BATCH * _SEQ
_NBLK = _N // 128


def _wid():
  return lax.axis_index("s") * 2 + lax.axis_index("c")


def _iota16():
  return lax.iota(jnp.int32, 16)


def _make_transpose():
  mesh = plsc.VectorSubcoreMesh(core_axis_name="c", subcore_axis_name="s")
  cols_per_w = 246

  @functools.partial(
      pl.kernel,
      mesh=mesh,
      out_type=jax.ShapeDtypeStruct((_VPAD, 128), jnp.float32),
      scratch_types=[
          pltpu.VMEM((_D, 128), jnp.float32),
          pltpu.VMEM((_D, 128), jnp.float32),
          pltpu.VMEM((128, 129), jnp.float32),
          pltpu.VMEM((128, 129), jnp.float32),
          pltpu.SemaphoreType.DMA,
          pltpu.SemaphoreType.DMA,
          pltpu.SemaphoreType.DMA,
          pltpu.SemaphoreType.DMA,
      ],
      compiler_params=pltpu.CompilerParams(
          use_tc_tiling_on_sc=True, disable_bounds_checks=True,
          needs_layout_passes=False),
  )
  def transpose_kernel(et_hbm, out_hbm, in0, in1, tb0, tb1, g0, g1, s0, s1):
    base = _wid() * cols_per_w

    def col(i):
      return jnp.minimum(base + i, _VT - 1)

    def fire_read(i, buf, sem):
      pltpu.async_copy(et_hbm.at[:, pl.ds(col(i) * 128, 128)], buf, sem)

    def wait_read(buf, sem):
      pltpu.make_async_copy(et_hbm.at[:, pl.ds(0, 128)], buf, sem).wait()

    def transpose(src, dst):
      rows = [_iota16() + j * _LANES for j in range(128 // _LANES)]

      @plsc.parallel_loop(0, _D, 1, unroll=8)
      def _(c):
        cv = jnp.full((16,), c, dtype=jnp.int32)
        for j in range(128 // _LANES):
          plsc.store_scatter(dst, [rows[j], cv],
                             src[c, pl.ds(j * _LANES, _LANES)])

    def fire_write(i, buf, sem):
      pltpu.async_copy(buf.at[:, pl.ds(0, 128)],
                       out_hbm.at[pl.ds(col(i) * 128, 128)], sem)

    def wait_write(buf, sem):
      pltpu.make_async_copy(buf.at[:, pl.ds(0, 128)],
                            out_hbm.at[pl.ds(0, 128)], sem).wait()

    fire_read(0, in0, g0)
    fire_read(1, in1, g1)

    def body(m, carry):
      i0 = 2 * m
      wait_read(in0, g0)

      @pl.when(m > 0)
      def _():
        wait_write(tb0, s0)

      transpose(in0, tb0)
      fire_write(i0, tb0, s0)

      @pl.when(m < cols_per_w // 2 - 1)
      def _():
        fire_read(i0 + 2, in0, g0)

      wait_read(in1, g1)

      @pl.when(m > 0)
      def _():
        wait_write(tb1, s1)

      transpose(in1, tb1)
      fire_write(i0 + 1, tb1, s1)

      @pl.when(m < cols_per_w // 2 - 1)
      def _():
        fire_read(i0 + 3, in1, g1)

      return carry

    lax.fori_loop(0, cols_per_w // 2, body, 0)
    wait_write(tb0, s0)
    wait_write(tb1, s1)

  return transpose_kernel


def _make_gather():
  mesh = plsc.VectorSubcoreMesh(core_axis_name="c", subcore_axis_name="s")
  blks_per_w = _NBLK // _NUM_WORKERS
  ring = 4

  @functools.partial(
      pl.kernel,
      mesh=mesh,
      out_type=jax.ShapeDtypeStruct((_SEQ, 8, 32, 8, 128), jnp.float32),
      scratch_types=(
          [pltpu.VMEM((blks_per_w, 128), jnp.int32)]
          + [pltpu.VMEM((128, _D), jnp.float32)] * ring
          + [pltpu.VMEM((_D, 129), jnp.float32)] * ring
          + [pltpu.SemaphoreType.DMA] * (2 * ring)
      ),
      compiler_params=pltpu.CompilerParams(
          use_tc_tiling_on_sc=False, needs_layout_passes=False),
  )
  def gather_kernel(ids_hbm, table_hbm, out_hbm, idx_v, *bufs):
    rb = bufs[0:ring]
    tb = bufs[ring:2 * ring]
    gsem = bufs[2 * ring:3 * ring]
    ssem = bufs[3 * ring:4 * ring]
    base = _wid() * blks_per_w

    pltpu.sync_copy(ids_hbm.at[pl.ds(base, blks_per_w)], idx_v)

    def fire_gather(i, u):
      pltpu.async_copy(table_hbm.at[idx_v.at[i]], rb[u], gsem[u])

    def wait_gather(u):
      pltpu.make_async_copy(table_hbm.at[idx_v.at[0]], rb[u],
                            gsem[u]).wait()

    def transpose(src, dst):
      rows = [_iota16() + j * _LANES for j in range(_D // _LANES)]

      @plsc.parallel_loop(0, 128, 1, unroll=8)
      def _(br):
        brv = jnp.full((16,), br, dtype=jnp.int32)
        for j in range(_D // _LANES):
          plsc.store_scatter(dst, [rows[j], brv],
                             src[br, pl.ds(j * _LANES, _LANES)])

    def fire_store(i, u):
      r = base + i
      s = r // 32
      k = r % 32
      for ft in range(8):
        pltpu.async_copy(tb[u].at[pl.ds(8 * ft, 8), pl.ds(0, 128)],
                         out_hbm.at[s, ft, k], ssem[u])

    def wait_store(u):
      for ft in range(8):
        pltpu.make_async_copy(tb[u].at[pl.ds(0, 8), pl.ds(0, 128)],
                              out_hbm.at[0, 0, 0], ssem[u]).wait()

    for u in range(ring):
      fire_gather(u, u)

    def body(m, carry):
      i0 = ring * m
      for u in range(ring):
        wait_gather(u)

        @pl.when(m > 0)
        def _():
          wait_store(u)

        transpose(rb[u], tb[u])
        fire_store(i0 + u, u)

        @pl.when(m < blks_per_w // ring - 1)
        def _():
          fire_gather(i0 + ring + u, u)

      return carry

    lax.fori_loop(0, blks_per_w // ring, body, 0)
    for u in range(ring):
      wait_store(u)

  return gather_kernel


def kernel(token_ids, E):
  ids = token_ids.T.reshape(_NBLK, 128).astype(jnp.int32) * 2
  table = _make_transpose()(E.T).reshape(2 * _VPAD, _D)
  out5 = _make_gather()(ids, table)
  return out5.transpose(2, 4, 0, 1, 3).reshape(_BATCH, _SEQ, _D)
